# baseline (speedup 1.0000x reference)
import jax
import jax.numpy as jnp
from jax import lax
from jax.experimental import pallas as pl
from jax.experimental.pallas import tpu as pltpu
from jax.experimental.pallas import tpu_sc as plsc

N = 1048576
NC = 2
NS = 16
NW = NC * NS
CHUNK = N // NW
LANES = 16
STEPS = CHUNK // LANES


def _dot_kernel(x_hbm, w_hbm, out_hbm, xv, wv, accbuf, sem_x, sem_w):
    cid = lax.axis_index("c")
    sid = lax.axis_index("s")
    wid = sid * NC + cid
    base = wid * CHUNK

    cp_x = pltpu.async_copy(x_hbm.at[pl.ds(base, CHUNK)], xv, sem_x)
    cp_w = pltpu.async_copy(w_hbm.at[pl.ds(base, CHUNK)], wv, sem_w)
    cp_x.wait()
    cp_w.wait()

    def body(i, acc):
        return acc + xv[pl.ds(i * LANES, LANES)] * wv[pl.ds(i * LANES, LANES)]

    acc = lax.fori_loop(0, STEPS, body, jnp.zeros((LANES,), jnp.float32))

    accbuf[...] = acc
    pltpu.sync_copy(accbuf, out_hbm.at[wid])


@jax.jit
def kernel(x, W_vals):
    xf = x.reshape(N)
    mesh = plsc.VectorSubcoreMesh(core_axis_name="c", subcore_axis_name="s")
    run = pl.kernel(
        _dot_kernel,
        out_type=jax.ShapeDtypeStruct((NW, LANES), jnp.float32),
        mesh=mesh,
        scratch_types=[
            pltpu.VMEM((CHUNK,), jnp.float32),
            pltpu.VMEM((CHUNK,), jnp.float32),
            pltpu.VMEM((LANES,), jnp.float32),
            pltpu.SemaphoreType.DMA,
            pltpu.SemaphoreType.DMA,
        ],
    )
    partials = run(xf, W_vals)
    total = jnp.sum(partials)
    return total * total


# speedup vs baseline: 1.1910x; 1.1910x over previous
import jax
import jax.numpy as jnp
from jax import lax
from jax.experimental import pallas as pl
from jax.experimental.pallas import tpu as pltpu
from jax.experimental.pallas import tpu_sc as plsc

N = 1048576
NC = 2
NS = 16
NW = NC * NS
CHUNK = N // NW
LANES = 16
STEPS = CHUNK // LANES


def _dot_kernel(x_hbm, w_hbm, out_hbm, xv, wv, accbuf, sem_x, sem_w):
    cid = lax.axis_index("c")
    sid = lax.axis_index("s")
    wid = sid * NC + cid
    base = wid * CHUNK

    cp_x = pltpu.async_copy(x_hbm.at[pl.ds(base, CHUNK)], xv, sem_x)
    cp_w = pltpu.async_copy(w_hbm.at[pl.ds(base, CHUNK)], wv, sem_w)
    cp_x.wait()
    cp_w.wait()

    NACC = 8
    zero = jnp.zeros((LANES,), jnp.float32)

    @plsc.parallel_loop(0, STEPS, step=NACC, unroll=2, carry=(zero,) * NACC)
    def accs(i, accs):
        base_i = i * LANES
        return tuple(
            a + xv[pl.ds(base_i + k * LANES, LANES)] * wv[pl.ds(base_i + k * LANES, LANES)]
            for k, a in enumerate(accs)
        )

    acc = zero
    for a in accs:
        acc = acc + a

    accbuf[...] = acc
    pltpu.sync_copy(accbuf, out_hbm.at[wid])


@jax.jit
def kernel(x, W_vals):
    xf = x.reshape(N)
    mesh = plsc.VectorSubcoreMesh(core_axis_name="c", subcore_axis_name="s")
    run = pl.kernel(
        _dot_kernel,
        out_type=jax.ShapeDtypeStruct((NW, LANES), jnp.float32),
        mesh=mesh,
        scratch_types=[
            pltpu.VMEM((CHUNK,), jnp.float32),
            pltpu.VMEM((CHUNK,), jnp.float32),
            pltpu.VMEM((LANES,), jnp.float32),
            pltpu.SemaphoreType.DMA,
            pltpu.SemaphoreType.DMA,
        ],
    )
    partials = run(xf, W_vals)
    total = jnp.sum(partials)
    return total * total
